# Initial kernel scaffold; baseline (speedup 1.0000x reference)
import jax
import jax.numpy as jnp
from jax.experimental import pallas as pl


def kernel(x, edge_index, batch, params):
    raise NotImplementedError("write your pallas kernel here")


# speedup vs baseline: 11.5417x; 11.5417x over previous
import functools

import jax
import jax.numpy as jnp
from jax import lax
from jax.experimental import pallas as pl
from jax.experimental.pallas import tpu as pltpu
from jax.experimental.pallas import tpu_sc as plsc

NN = 10000
NE = 320000
HID = 128
EPS = 1e-5
CH = 128
NW = 32
NCHUNK = NE // CH
ROWS_PT = 624
BLK = 1000
NB = NN // BLK
DEGW = 128


def _row_partition_copy(s, fn):
    @pl.when(s == 0)
    def _():
        fn(0, 640)

    @pl.when(s > 0)
    def _():
        fn(16 + s * ROWS_PT, ROWS_PT)


def _sc_agg_body(table, src, dst, zeros, out, sidx, didx, rows, acc):
    c = lax.axis_index("c")
    s = lax.axis_index("s")
    w = s * 2 + c
    _row_partition_copy(s, lambda base, n: pltpu.sync_copy(
        zeros.at[pl.ds(base, n)], acc.at[pl.ds(base, n)]))
    plsc.subcore_barrier()

    def body(j, carry):
        cid = j * NW + w

        @pl.when(cid < NCHUNK)
        def _():
            base = cid * CH
            pltpu.sync_copy(src.at[pl.ds(base, CH)], sidx)
            pltpu.sync_copy(dst.at[pl.ds(base, CH)], didx)
            pltpu.sync_copy(table.at[sidx], rows)
            pltpu.sync_copy(rows, acc.at[didx], add=True)
        return carry

    lax.fori_loop(0, (NCHUNK + NW - 1) // NW, body, 0)
    plsc.subcore_barrier()
    _row_partition_copy(s, lambda base, n: pltpu.sync_copy(
        acc.at[pl.ds(base, n)], out.at[c, pl.ds(base, n)]))


def _sc_deg_body(dst, zeros, ones, out, didx, ones_v, acc):
    c = lax.axis_index("c")
    s = lax.axis_index("s")
    w = s * 2 + c
    pltpu.sync_copy(ones, ones_v)
    _row_partition_copy(s, lambda base, n: pltpu.sync_copy(
        zeros.at[pl.ds(base, n)], acc.at[pl.ds(base, n)]))
    plsc.subcore_barrier()

    def body(j, carry):
        cid = j * NW + w

        @pl.when(cid < NCHUNK)
        def _():
            pltpu.sync_copy(dst.at[pl.ds(cid * CH, CH)], didx)
            pltpu.sync_copy(ones_v, acc.at[didx], add=True)
        return carry

    lax.fori_loop(0, (NCHUNK + NW - 1) // NW, body, 0)
    plsc.subcore_barrier()
    _row_partition_copy(s, lambda base, n: pltpu.sync_copy(
        acc.at[pl.ds(base, n)], out.at[c, pl.ds(base, n)]))


@functools.lru_cache(maxsize=None)
def _build_sc_kernels():
    mesh = plsc.VectorSubcoreMesh(core_axis_name="c", subcore_axis_name="s")
    agg = pl.kernel(
        _sc_agg_body,
        out_type=jax.ShapeDtypeStruct((2, NN, HID), jnp.float32),
        mesh=mesh,
        scratch_types=[
            pltpu.VMEM((CH,), jnp.int32),
            pltpu.VMEM((CH,), jnp.int32),
            pltpu.VMEM((CH, HID), jnp.float32),
            pltpu.VMEM_SHARED((NN, HID), jnp.float32),
        ],
    )
    deg = pl.kernel(
        _sc_deg_body,
        out_type=jax.ShapeDtypeStruct((2, NN, DEGW), jnp.float32),
        mesh=mesh,
        scratch_types=[
            pltpu.VMEM((CH,), jnp.int32),
            pltpu.VMEM((CH, DEGW), jnp.float32),
            pltpu.VMEM_SHARED((NN, DEGW), jnp.float32),
        ],
    )
    return agg, deg


def _in_body(x_ref, w_ref, b_ref, o_ref):
    o_ref[...] = jnp.maximum(
        jnp.dot(x_ref[...], w_ref[...], preferred_element_type=jnp.float32)
        + b_ref[...], 0.0)


def _prep_body(degp_ref, h_ref, w_ref, hwp_ref, dinv_ref):
    deg = degp_ref[0, :, 0:1] + degp_ref[1, :, 0:1] + 1.0
    dinvb = jnp.broadcast_to(lax.rsqrt(jnp.maximum(deg, 1.0)), (BLK, HID))
    dinv_ref[...] = dinvb
    hwp_ref[...] = dinvb * jnp.dot(h_ref[...], w_ref[...],
                                   preferred_element_type=jnp.float32)


def _layer_a_body(p_ref, hwp_ref, dinv_ref, b_ref, y_ref, st_ref):
    i = pl.program_id(0)
    y = dinv_ref[...] * (p_ref[0] + p_ref[1] + hwp_ref[...]) + b_ref[...]
    y_ref[...] = y

    @pl.when(i == 0)
    def _():
        st_ref[...] = jnp.zeros_like(st_ref)

    st_ref[...] = st_ref[...] + jnp.sum(y, axis=0, keepdims=True)


def _var_body(y_ref, sum_ref, v_ref):
    i = pl.program_id(0)
    d = y_ref[...] - sum_ref[...] / NN

    @pl.when(i == 0)
    def _():
        v_ref[...] = jnp.zeros_like(v_ref)

    v_ref[...] = v_ref[...] + jnp.sum(d * d, axis=0, keepdims=True)


def _layer_b_body(y_ref, sum_ref, v_ref, g_ref, be_ref, w_ref, dinv_ref,
                  o_ref):
    mean = sum_ref[...] / NN
    var = v_ref[...] / NN
    h = jnp.maximum((y_ref[...] - mean) * lax.rsqrt(var + EPS) * g_ref[...]
                    + be_ref[...], 0.0)
    o_ref[...] = dinv_ref[...] * jnp.dot(h, w_ref[...],
                                         preferred_element_type=jnp.float32)


def _final_body(y_ref, sum_ref, v_ref, g_ref, be_ref, w1_ref, b1_ref, w2_ref,
                b2_ref, o_ref, acc_ref):
    i = pl.program_id(0)
    mean = sum_ref[...] / NN
    var = v_ref[...] / NN
    h = jnp.maximum((y_ref[...] - mean) * lax.rsqrt(var + EPS) * g_ref[...]
                    + be_ref[...], 0.0)

    @pl.when(i == 0)
    def _():
        acc_ref[...] = jnp.zeros_like(acc_ref)

    acc_ref[...] = acc_ref[...] + jnp.sum(h, axis=0, keepdims=True)

    @pl.when(i == NB - 1)
    def _():
        pooled = acc_ref[...] / NN
        z = jnp.maximum(
            jnp.dot(pooled, w1_ref[...], preferred_element_type=jnp.float32)
            + b1_ref[...], 0.0)
        o_ref[...] = jnp.dot(z, w2_ref[...],
                             preferred_element_type=jnp.float32) + b2_ref[...]


def _row_spec(shape):
    nd = len(shape)
    if nd == 2:
        return pl.BlockSpec((BLK, shape[1]), lambda i: (i, 0))
    return pl.BlockSpec((shape[0], BLK, shape[2]), lambda i: (0, i, 0))


def _full_spec(shape):
    return pl.BlockSpec(shape, lambda i: tuple(0 for _ in shape))


def _tc_call(body, ins, outs, scratch=()):
    in_specs = [_row_spec(a.shape) if m == "row" else _full_spec(a.shape)
                for a, m in ins]
    out_shapes = [jax.ShapeDtypeStruct(s, jnp.float32) for s, _ in outs]
    out_specs = [_row_spec(s) if m == "row" else _full_spec(s)
                 for s, m in outs]
    return pl.pallas_call(
        body,
        grid=(NB,),
        in_specs=in_specs,
        out_specs=out_specs[0] if len(outs) == 1 else out_specs,
        out_shape=out_shapes[0] if len(outs) == 1 else out_shapes,
        scratch_shapes=list(scratch),
    )(*[a for a, _ in ins])


def kernel(x, edge_index, batch, params):
    _sc_agg, _sc_deg = _build_sc_kernels()
    src = edge_index[0]
    dst = edge_index[1]
    zeros_h = jnp.zeros((NN, HID), jnp.float32)

    in_b = params['in_b'].reshape(1, HID)
    h = _tc_call(_in_body,
                 [(x, "row"), (params['in_W'], "full"), (in_b, "full")],
                 [((NN, HID), "row")])

    degp = _sc_deg(dst, zeros_h, jnp.ones((CH, DEGW), jnp.float32))
    hwp, dinvb = _tc_call(
        _prep_body,
        [(degp, "row"), (h, "row"), (params['conv_W'][0], "full")],
        [((NN, HID), "row"), ((NN, HID), "row")])

    out = None
    for i in range(4):
        parts = _sc_agg(hwp, src, dst, zeros_h)
        b2 = params['conv_b'][i].reshape(1, HID)
        y, ysum = _tc_call(
            _layer_a_body,
            [(parts, "row"), (hwp, "row"), (dinvb, "row"), (b2, "full")],
            [((NN, HID), "row"), ((1, HID), "full")])
        vsum = _tc_call(
            _var_body,
            [(y, "row"), (ysum, "full")],
            [((1, HID), "full")])
        g2 = params['bn_gamma'][i].reshape(1, HID)
        be2 = params['bn_beta'][i].reshape(1, HID)
        if i < 3:
            hwp = _tc_call(
                _layer_b_body,
                [(y, "row"), (ysum, "full"), (vsum, "full"), (g2, "full"),
                 (be2, "full"), (params['conv_W'][i + 1], "full"),
                 (dinvb, "row")],
                [((NN, HID), "row")])
        else:
            out = _tc_call(
                _final_body,
                [(y, "row"), (ysum, "full"), (vsum, "full"), (g2, "full"),
                 (be2, "full"), (params['fc1_W'], "full"),
                 (params['fc1_b'].reshape(1, HID // 2), "full"),
                 (params['fc2_W'], "full"),
                 (params['fc2_b'].reshape(1, 10), "full")],
                [((1, 10), "full")],
                scratch=[pltpu.VMEM((1, HID), jnp.float32)])
    return out


# speedup vs baseline: 16.7228x; 1.4489x over previous
import functools

import jax
import jax.numpy as jnp
from jax import lax
from jax.experimental import pallas as pl
from jax.experimental.pallas import tpu as pltpu
from jax.experimental.pallas import tpu_sc as plsc

NN = 10000
NE = 320000
HID = 128
EPS = 1e-5
CH = 128
NW = 32
NCHUNK = NE // CH
ROWS_PT = 624
BLK = 1000
NB = NN // BLK
DEGW = 128


def _row_partition_copy(s, fn):
    @pl.when(s == 0)
    def _():
        fn(0, 640)

    @pl.when(s > 0)
    def _():
        fn(16 + s * ROWS_PT, ROWS_PT)


def _sc_agg_body(table, src, dst, zeros, out,
                 s0, d0, r0, s1, d1, r1, sem0, sem1, acc):
    c = lax.axis_index("c")
    s = lax.axis_index("s")
    w = s * 2 + c
    _row_partition_copy(s, lambda base, n: pltpu.sync_copy(
        zeros.at[pl.ds(base, n)], acc.at[pl.ds(base, n)]))
    plsc.subcore_barrier()

    sb, db, rb, sems = (s0, s1), (d0, d1), (r0, r1), (sem0, sem1)
    nt = (NCHUNK + NW - 1) // NW

    def load_start(t, b):
        cid = t * NW + w

        @pl.when(cid < NCHUNK)
        def _():
            base = cid * CH
            pltpu.sync_copy(src.at[pl.ds(base, CH)], sb[b])
            pltpu.sync_copy(dst.at[pl.ds(base, CH)], db[b])
            pltpu.async_copy(table.at[sb[b]], rb[b], sems[b])

    def consume(t, b):
        cid = t * NW + w

        @pl.when(cid < NCHUNK)
        def _():
            pltpu.make_async_copy(table.at[sb[b]], rb[b], sems[b]).wait()
            pltpu.sync_copy(rb[b], acc.at[db[b]], add=True)

    load_start(0, 0)

    def body(j2, carry):
        for b in range(2):
            t = j2 * 2 + b
            load_start(t + 1, 1 - b)
            consume(t, b)
        return carry

    lax.fori_loop(0, (nt + 1) // 2, body, 0)
    plsc.subcore_barrier()
    _row_partition_copy(s, lambda base, n: pltpu.sync_copy(
        acc.at[pl.ds(base, n)], out.at[c, pl.ds(base, n)]))


def _sc_deg_body(dst, zeros, ones, out, didx, ones_v, acc):
    c = lax.axis_index("c")
    s = lax.axis_index("s")
    w = s * 2 + c
    pltpu.sync_copy(ones, ones_v)
    _row_partition_copy(s, lambda base, n: pltpu.sync_copy(
        zeros.at[pl.ds(base, n)], acc.at[pl.ds(base, n)]))
    plsc.subcore_barrier()

    def body(j, carry):
        cid = j * NW + w

        @pl.when(cid < NCHUNK)
        def _():
            pltpu.sync_copy(dst.at[pl.ds(cid * CH, CH)], didx)
            pltpu.sync_copy(ones_v, acc.at[didx], add=True)
        return carry

    lax.fori_loop(0, (NCHUNK + NW - 1) // NW, body, 0)
    plsc.subcore_barrier()
    _row_partition_copy(s, lambda base, n: pltpu.sync_copy(
        acc.at[pl.ds(base, n)], out.at[c, pl.ds(base, n)]))


@functools.lru_cache(maxsize=None)
def _build_sc_kernels():
    mesh = plsc.VectorSubcoreMesh(core_axis_name="c", subcore_axis_name="s")
    agg = pl.kernel(
        _sc_agg_body,
        out_type=jax.ShapeDtypeStruct((2, NN, HID), jnp.float32),
        mesh=mesh,
        scratch_types=[
            pltpu.VMEM((CH,), jnp.int32),
            pltpu.VMEM((CH,), jnp.int32),
            pltpu.VMEM((CH, HID), jnp.float32),
            pltpu.VMEM((CH,), jnp.int32),
            pltpu.VMEM((CH,), jnp.int32),
            pltpu.VMEM((CH, HID), jnp.float32),
            pltpu.SemaphoreType.DMA,
            pltpu.SemaphoreType.DMA,
            pltpu.VMEM_SHARED((NN, HID), jnp.float32),
        ],
    )
    deg = pl.kernel(
        _sc_deg_body,
        out_type=jax.ShapeDtypeStruct((2, NN, DEGW), jnp.float32),
        mesh=mesh,
        scratch_types=[
            pltpu.VMEM((CH,), jnp.int32),
            pltpu.VMEM((CH, DEGW), jnp.float32),
            pltpu.VMEM_SHARED((NN, DEGW), jnp.float32),
        ],
    )
    return agg, deg


def _in_body(x_ref, w_ref, b_ref, o_ref):
    o_ref[...] = jnp.maximum(
        jnp.dot(x_ref[...], w_ref[...], preferred_element_type=jnp.float32)
        + b_ref[...], 0.0)


def _prep_body(degp_ref, h_ref, w_ref, hwp_ref, dinv_ref):
    deg = degp_ref[0, :, 0:1] + degp_ref[1, :, 0:1] + 1.0
    dinvb = jnp.broadcast_to(lax.rsqrt(jnp.maximum(deg, 1.0)), (BLK, HID))
    dinv_ref[...] = dinvb
    hwp_ref[...] = dinvb * jnp.dot(h_ref[...], w_ref[...],
                                   preferred_element_type=jnp.float32)


def _layer_a_body(p_ref, hwp_ref, dinv_ref, b_ref, y_ref, st_ref):
    i = pl.program_id(0)
    y = dinv_ref[...] * (p_ref[0] + p_ref[1] + hwp_ref[...]) + b_ref[...]
    y_ref[...] = y

    @pl.when(i == 0)
    def _():
        st_ref[...] = jnp.zeros_like(st_ref)

    st_ref[...] = st_ref[...] + jnp.sum(y, axis=0, keepdims=True)


def _var_body(y_ref, sum_ref, v_ref):
    i = pl.program_id(0)
    d = y_ref[...] - sum_ref[...] / NN

    @pl.when(i == 0)
    def _():
        v_ref[...] = jnp.zeros_like(v_ref)

    v_ref[...] = v_ref[...] + jnp.sum(d * d, axis=0, keepdims=True)


def _layer_b_body(y_ref, sum_ref, v_ref, g_ref, be_ref, w_ref, dinv_ref,
                  o_ref):
    mean = sum_ref[...] / NN
    var = v_ref[...] / NN
    h = jnp.maximum((y_ref[...] - mean) * lax.rsqrt(var + EPS) * g_ref[...]
                    + be_ref[...], 0.0)
    o_ref[...] = dinv_ref[...] * jnp.dot(h, w_ref[...],
                                         preferred_element_type=jnp.float32)


def _final_body(y_ref, sum_ref, v_ref, g_ref, be_ref, w1_ref, b1_ref, w2_ref,
                b2_ref, o_ref, acc_ref):
    i = pl.program_id(0)
    mean = sum_ref[...] / NN
    var = v_ref[...] / NN
    h = jnp.maximum((y_ref[...] - mean) * lax.rsqrt(var + EPS) * g_ref[...]
                    + be_ref[...], 0.0)

    @pl.when(i == 0)
    def _():
        acc_ref[...] = jnp.zeros_like(acc_ref)

    acc_ref[...] = acc_ref[...] + jnp.sum(h, axis=0, keepdims=True)

    @pl.when(i == NB - 1)
    def _():
        pooled = acc_ref[...] / NN
        z = jnp.maximum(
            jnp.dot(pooled, w1_ref[...], preferred_element_type=jnp.float32)
            + b1_ref[...], 0.0)
        o_ref[...] = jnp.dot(z, w2_ref[...],
                             preferred_element_type=jnp.float32) + b2_ref[...]


def _row_spec(shape):
    nd = len(shape)
    if nd == 2:
        return pl.BlockSpec((BLK, shape[1]), lambda i: (i, 0))
    return pl.BlockSpec((shape[0], BLK, shape[2]), lambda i: (0, i, 0))


def _full_spec(shape):
    return pl.BlockSpec(shape, lambda i: tuple(0 for _ in shape))


def _tc_call(body, ins, outs, scratch=()):
    in_specs = [_row_spec(a.shape) if m == "row" else _full_spec(a.shape)
                for a, m in ins]
    out_shapes = [jax.ShapeDtypeStruct(s, jnp.float32) for s, _ in outs]
    out_specs = [_row_spec(s) if m == "row" else _full_spec(s)
                 for s, m in outs]
    return pl.pallas_call(
        body,
        grid=(NB,),
        in_specs=in_specs,
        out_specs=out_specs[0] if len(outs) == 1 else out_specs,
        out_shape=out_shapes[0] if len(outs) == 1 else out_shapes,
        scratch_shapes=list(scratch),
    )(*[a for a, _ in ins])


def kernel(x, edge_index, batch, params):
    _sc_agg, _sc_deg = _build_sc_kernels()
    src = edge_index[0]
    dst = edge_index[1]
    zeros_h = jnp.zeros((NN, HID), jnp.float32)

    in_b = params['in_b'].reshape(1, HID)
    h = _tc_call(_in_body,
                 [(x, "row"), (params['in_W'], "full"), (in_b, "full")],
                 [((NN, HID), "row")])

    degp = _sc_deg(dst, zeros_h, jnp.ones((CH, DEGW), jnp.float32))
    hwp, dinvb = _tc_call(
        _prep_body,
        [(degp, "row"), (h, "row"), (params['conv_W'][0], "full")],
        [((NN, HID), "row"), ((NN, HID), "row")])

    out = None
    for i in range(4):
        parts = _sc_agg(hwp, src, dst, zeros_h)
        b2 = params['conv_b'][i].reshape(1, HID)
        y, ysum = _tc_call(
            _layer_a_body,
            [(parts, "row"), (hwp, "row"), (dinvb, "row"), (b2, "full")],
            [((NN, HID), "row"), ((1, HID), "full")])
        vsum = _tc_call(
            _var_body,
            [(y, "row"), (ysum, "full")],
            [((1, HID), "full")])
        g2 = params['bn_gamma'][i].reshape(1, HID)
        be2 = params['bn_beta'][i].reshape(1, HID)
        if i < 3:
            hwp = _tc_call(
                _layer_b_body,
                [(y, "row"), (ysum, "full"), (vsum, "full"), (g2, "full"),
                 (be2, "full"), (params['conv_W'][i + 1], "full"),
                 (dinvb, "row")],
                [((NN, HID), "row")])
        else:
            out = _tc_call(
                _final_body,
                [(y, "row"), (ysum, "full"), (vsum, "full"), (g2, "full"),
                 (be2, "full"), (params['fc1_W'], "full"),
                 (params['fc1_b'].reshape(1, HID // 2), "full"),
                 (params['fc2_W'], "full"),
                 (params['fc2_b'].reshape(1, 10), "full")],
                [((1, 10), "full")],
                scratch=[pltpu.VMEM((1, HID), jnp.float32)])
    return out


# speedup vs baseline: 19.5662x; 1.1700x over previous
import functools

import jax
import jax.numpy as jnp
from jax import lax
from jax.experimental import pallas as pl
from jax.experimental.pallas import tpu as pltpu
from jax.experimental.pallas import tpu_sc as plsc

NN = 10000
NE = 320000
HID = 128
EPS = 1e-5
CH = 128
NW = 32
NCHUNK = NE // CH
ROWS_PT = 624
BLK = 1000
NB = NN // BLK
DEGW = 128


def _row_partition_copy(s, fn):
    @pl.when(s == 0)
    def _():
        fn(0, 640)

    @pl.when(s > 0)
    def _():
        fn(16 + s * ROWS_PT, ROWS_PT)


CPT = 80
BG = 8
NGRP = CPT // BG


def _sc_agg_body(src2d, dst2d, table, zeros, out,
                 sb0, db0, sb1, db1, r0, r1, sem0, sem1, acc):
    c = lax.axis_index("c")
    s = lax.axis_index("s")
    w = s * 2 + c
    _row_partition_copy(s, lambda base, n: pltpu.sync_copy(
        zeros.at[pl.ds(base, n)], acc.at[pl.ds(base, n)]))
    plsc.subcore_barrier()

    sblk, dblk = (sb0, sb1), (db0, db1)
    rb, sems = (r0, r1), (sem0, sem1)
    start = w * CPT
    count = jnp.where(w == NW - 1, NCHUNK - (NW - 1) * CPT, CPT)

    def copy_block(g, gb):
        @pl.when(g * BG < count)
        def _():
            pltpu.sync_copy(src2d.at[pl.ds((start + g * BG), BG)], sblk[gb])
            pltpu.sync_copy(dst2d.at[pl.ds((start + g * BG), BG)], dblk[gb])

    def step(cl, gb, k):
        b = k % 2
        nk, ngb = (k + 1, gb) if k < BG - 1 else (0, 1 - gb)

        @pl.when(cl + 1 < count)
        def _():
            pltpu.async_copy(table.at[sblk[ngb].at[nk]], rb[1 - b],
                             sems[1 - b])

        @pl.when(cl < count)
        def _():
            pltpu.make_async_copy(table.at[sblk[gb].at[k]], rb[b],
                                  sems[b]).wait()
            pltpu.sync_copy(rb[b], acc.at[dblk[gb].at[k]], add=True)

    copy_block(0, 0)
    pltpu.async_copy(table.at[sblk[0].at[0]], rb[0], sems[0])

    def body(g2, carry):
        for gb in range(2):
            g = g2 * 2 + gb
            copy_block(g + 1, 1 - gb)
            for k in range(BG):
                step(g * BG + k, gb, k)
        return carry

    lax.fori_loop(0, NGRP // 2, body, 0)
    plsc.subcore_barrier()
    _row_partition_copy(s, lambda base, n: pltpu.sync_copy(
        acc.at[pl.ds(base, n)], out.at[c, pl.ds(base, n)]))


def _sc_deg_body(dst, zeros, ones, out, didx, ones_v, acc):
    c = lax.axis_index("c")
    s = lax.axis_index("s")
    w = s * 2 + c
    pltpu.sync_copy(ones, ones_v)
    _row_partition_copy(s, lambda base, n: pltpu.sync_copy(
        zeros.at[pl.ds(base, n)], acc.at[pl.ds(base, n)]))
    plsc.subcore_barrier()

    def body(j, carry):
        cid = j * NW + w

        @pl.when(cid < NCHUNK)
        def _():
            pltpu.sync_copy(dst.at[pl.ds(cid * CH, CH)], didx)
            pltpu.sync_copy(ones_v, acc.at[didx], add=True)
        return carry

    lax.fori_loop(0, (NCHUNK + NW - 1) // NW, body, 0)
    plsc.subcore_barrier()
    _row_partition_copy(s, lambda base, n: pltpu.sync_copy(
        acc.at[pl.ds(base, n)], out.at[c, pl.ds(base, n)]))


@functools.lru_cache(maxsize=None)
def _build_sc_kernels():
    mesh = plsc.VectorSubcoreMesh(core_axis_name="c", subcore_axis_name="s")
    agg = pl.kernel(
        _sc_agg_body,
        out_type=jax.ShapeDtypeStruct((2, NN, HID), jnp.float32),
        mesh=mesh,
        scratch_types=[
            pltpu.VMEM((BG, CH), jnp.int32),
            pltpu.VMEM((BG, CH), jnp.int32),
            pltpu.VMEM((BG, CH), jnp.int32),
            pltpu.VMEM((BG, CH), jnp.int32),
            pltpu.VMEM((CH, HID), jnp.float32),
            pltpu.VMEM((CH, HID), jnp.float32),
            pltpu.SemaphoreType.DMA,
            pltpu.SemaphoreType.DMA,
            pltpu.VMEM_SHARED((NN, HID), jnp.float32),
        ],
    )
    deg = pl.kernel(
        _sc_deg_body,
        out_type=jax.ShapeDtypeStruct((2, NN, DEGW), jnp.float32),
        mesh=mesh,
        scratch_types=[
            pltpu.VMEM((CH,), jnp.int32),
            pltpu.VMEM((CH, DEGW), jnp.float32),
            pltpu.VMEM_SHARED((NN, DEGW), jnp.float32),
        ],
    )
    return agg, deg


def _in_body(x_ref, w_ref, b_ref, o_ref):
    o_ref[...] = jnp.maximum(
        jnp.dot(x_ref[...], w_ref[...], preferred_element_type=jnp.float32)
        + b_ref[...], 0.0)


def _prep_body(degp_ref, h_ref, w_ref, hwp_ref, dinv_ref):
    deg = degp_ref[0, :, 0:1] + degp_ref[1, :, 0:1] + 1.0
    dinvb = jnp.broadcast_to(lax.rsqrt(jnp.maximum(deg, 1.0)), (BLK, HID))
    dinv_ref[...] = dinvb
    hwp_ref[...] = dinvb * jnp.dot(h_ref[...], w_ref[...],
                                   preferred_element_type=jnp.float32)


def _layer_a_body(p_ref, hwp_ref, dinv_ref, b_ref, y_ref, st_ref):
    i = pl.program_id(0)
    y = dinv_ref[...] * (p_ref[0] + p_ref[1] + hwp_ref[...]) + b_ref[...]
    y_ref[...] = y

    @pl.when(i == 0)
    def _():
        st_ref[...] = jnp.zeros_like(st_ref)

    st_ref[...] = st_ref[...] + jnp.sum(y, axis=0, keepdims=True)


def _var_body(y_ref, sum_ref, v_ref):
    i = pl.program_id(0)
    d = y_ref[...] - sum_ref[...] / NN

    @pl.when(i == 0)
    def _():
        v_ref[...] = jnp.zeros_like(v_ref)

    v_ref[...] = v_ref[...] + jnp.sum(d * d, axis=0, keepdims=True)


def _layer_b_body(y_ref, sum_ref, v_ref, g_ref, be_ref, w_ref, dinv_ref,
                  o_ref):
    mean = sum_ref[...] / NN
    var = v_ref[...] / NN
    h = jnp.maximum((y_ref[...] - mean) * lax.rsqrt(var + EPS) * g_ref[...]
                    + be_ref[...], 0.0)
    o_ref[...] = dinv_ref[...] * jnp.dot(h, w_ref[...],
                                         preferred_element_type=jnp.float32)


def _final_body(y_ref, sum_ref, v_ref, g_ref, be_ref, w1_ref, b1_ref, w2_ref,
                b2_ref, o_ref, acc_ref):
    i = pl.program_id(0)
    mean = sum_ref[...] / NN
    var = v_ref[...] / NN
    h = jnp.maximum((y_ref[...] - mean) * lax.rsqrt(var + EPS) * g_ref[...]
                    + be_ref[...], 0.0)

    @pl.when(i == 0)
    def _():
        acc_ref[...] = jnp.zeros_like(acc_ref)

    acc_ref[...] = acc_ref[...] + jnp.sum(h, axis=0, keepdims=True)

    @pl.when(i == NB - 1)
    def _():
        pooled = acc_ref[...] / NN
        z = jnp.maximum(
            jnp.dot(pooled, w1_ref[...], preferred_element_type=jnp.float32)
            + b1_ref[...], 0.0)
        o_ref[...] = jnp.dot(z, w2_ref[...],
                             preferred_element_type=jnp.float32) + b2_ref[...]


def _row_spec(shape):
    nd = len(shape)
    if nd == 2:
        return pl.BlockSpec((BLK, shape[1]), lambda i: (i, 0))
    return pl.BlockSpec((shape[0], BLK, shape[2]), lambda i: (0, i, 0))


def _full_spec(shape):
    return pl.BlockSpec(shape, lambda i: tuple(0 for _ in shape))


def _tc_call(body, ins, outs, scratch=()):
    in_specs = [_row_spec(a.shape) if m == "row" else _full_spec(a.shape)
                for a, m in ins]
    out_shapes = [jax.ShapeDtypeStruct(s, jnp.float32) for s, _ in outs]
    out_specs = [_row_spec(s) if m == "row" else _full_spec(s)
                 for s, m in outs]
    return pl.pallas_call(
        body,
        grid=(NB,),
        in_specs=in_specs,
        out_specs=out_specs[0] if len(outs) == 1 else out_specs,
        out_shape=out_shapes[0] if len(outs) == 1 else out_shapes,
        scratch_shapes=list(scratch),
    )(*[a for a, _ in ins])


PAD_CHUNK = 2504


def kernel(x, edge_index, batch, params):
    _sc_agg, _sc_deg = _build_sc_kernels()
    src = edge_index[0]
    dst = edge_index[1]
    src2d = jnp.zeros((PAD_CHUNK, CH), jnp.int32).at[:NCHUNK].set(
        src.reshape(NCHUNK, CH))
    dst2d = jnp.zeros((PAD_CHUNK, CH), jnp.int32).at[:NCHUNK].set(
        dst.reshape(NCHUNK, CH))
    zeros_h = jnp.zeros((NN, HID), jnp.float32)

    in_b = params['in_b'].reshape(1, HID)
    h = _tc_call(_in_body,
                 [(x, "row"), (params['in_W'], "full"), (in_b, "full")],
                 [((NN, HID), "row")])

    degp = _sc_deg(dst, zeros_h, jnp.ones((CH, DEGW), jnp.float32))
    hwp, dinvb = _tc_call(
        _prep_body,
        [(degp, "row"), (h, "row"), (params['conv_W'][0], "full")],
        [((NN, HID), "row"), ((NN, HID), "row")])

    out = None
    for i in range(4):
        parts = _sc_agg(src2d, dst2d, hwp, zeros_h)
        b2 = params['conv_b'][i].reshape(1, HID)
        y, ysum = _tc_call(
            _layer_a_body,
            [(parts, "row"), (hwp, "row"), (dinvb, "row"), (b2, "full")],
            [((NN, HID), "row"), ((1, HID), "full")])
        vsum = _tc_call(
            _var_body,
            [(y, "row"), (ysum, "full")],
            [((1, HID), "full")])
        g2 = params['bn_gamma'][i].reshape(1, HID)
        be2 = params['bn_beta'][i].reshape(1, HID)
        if i < 3:
            hwp = _tc_call(
                _layer_b_body,
                [(y, "row"), (ysum, "full"), (vsum, "full"), (g2, "full"),
                 (be2, "full"), (params['conv_W'][i + 1], "full"),
                 (dinvb, "row")],
                [((NN, HID), "row")])
        else:
            out = _tc_call(
                _final_body,
                [(y, "row"), (ysum, "full"), (vsum, "full"), (g2, "full"),
                 (be2, "full"), (params['fc1_W'], "full"),
                 (params['fc1_b'].reshape(1, HID // 2), "full"),
                 (params['fc2_W'], "full"),
                 (params['fc2_b'].reshape(1, 10), "full")],
                [((1, 10), "full")],
                scratch=[pltpu.VMEM((1, HID), jnp.float32)])
    return out
